# baseline (speedup 1.0000x reference)
import functools

import jax
import jax.numpy as jnp
from jax import lax
from jax.experimental import pallas as pl
from jax.experimental.pallas import tpu as pltpu
from jax.experimental.pallas import tpu_sc as plsc

_N = 128
_K = 16


def _pre_body(d_ref, dt_ref, sr_ref, sc_ref, nm_ref, nmt_ref,
              g_ref, h_ref, gt_ref, ht_ref, at_ref, mt_ref, av_ref):
    n = _N
    D = d_ref[...]
    DT = dt_ref[...]
    s_row = sr_ref[...]
    s_col = sc_ref[...]
    ii = lax.broadcasted_iota(jnp.int32, (n, n), 0)
    jj = lax.broadcasted_iota(jnp.int32, (n, n), 1)
    diag = ii == jj

    M = jnp.zeros((n, n), jnp.bool_)
    MT = jnp.zeros((n, n), jnp.bool_)
    for k in range(_K):
        col_k = nm_ref[:, k][:, None]
        row_k = nmt_ref[k, :][None, :]
        M = M | (col_k == jj)
        MT = MT | (row_k == ii)
    Mf = M.astype(jnp.float32)
    MTf = MT.astype(jnp.float32)

    NB = jnp.tanh((0.7 * D + 0.3 * s_row + 0.5) * 0.5)
    NBT = jnp.tanh((0.7 * DT + 0.3 * s_col + 0.5) * 0.5)
    CNT = jnp.dot(Mf, MTf, precision=lax.Precision.HIGHEST)
    cnt_pos = CNT > 0.0
    hval = jnp.where(cnt_pos, 0.8 / jnp.maximum(CNT, 1.0), 0.0)

    A = Mf * jnp.where(diag, 1.0, NB)
    at_ref[...] = MTf * jnp.where(diag, 1.0, NBT)
    mt_ref[...] = MTf

    F = jnp.broadcast_to(0.5 * s_row, (n, n))
    FT = jnp.broadcast_to(0.5 * s_col, (n, n))
    g_ref[...] = jnp.where(diag, 1.0, jnp.where(M, NB, jnp.where(cnt_pos, 0.0, F)))
    gt_ref[...] = jnp.where(diag, 1.0, jnp.where(MT, NBT, jnp.where(cnt_pos, 0.0, FT)))
    h_ref[...] = jnp.where(diag | M, 0.0, hval)
    ht_ref[...] = jnp.where(diag | MT, 0.0, hval)

    av_cols = []
    for k in range(_K):
        col_k = nm_ref[:, k][:, None]
        dup = jnp.zeros((n, 1), jnp.bool_)
        for t in range(k):
            dup = dup | (nm_ref[:, t][:, None] == col_k)
        hit = (col_k == jj).astype(jnp.float32)
        val = jnp.sum(A * hit, axis=1, keepdims=True)
        av_cols.append(jnp.where(dup, 0.0, val))
    av_ref[...] = jnp.concatenate(av_cols, axis=1)


def _precompute(dynamic_re, static_re, neighbor_matrix):
    f32 = jnp.float32
    shp = jax.ShapeDtypeStruct
    return pl.pallas_call(
        _pre_body,
        out_shape=(
            shp((_N, _N), f32),
            shp((_N, _N), f32),
            shp((_N, _N), f32),
            shp((_N, _N), f32),
            shp((_N, _N), f32),
            shp((_N, _N), f32),
            shp((_N, _K), f32),
        ),
    )(dynamic_re, dynamic_re.T, static_re.reshape(1, _N),
      static_re.reshape(_N, 1), neighbor_matrix, neighbor_matrix.T)


def _sc_body(gh_h, at_h, mt_h, nm_h, av_h, z_h, out_h,
             at_v, mt_v, nm_v, av_v, rep_v, ghb_v, gh_s):
    cid = lax.axis_index("c")
    sid = lax.axis_index("s")

    @pl.when(jnp.logical_and(cid == 0, sid == 0))
    def _work():
        pltpu.sync_copy(gh_h, gh_s)
        pltpu.sync_copy(at_h, at_v)
        pltpu.sync_copy(mt_h, mt_v)
        pltpu.sync_copy(nm_h, nm_v)
        pltpu.sync_copy(av_h, av_v)
        pltpu.sync_copy(z_h, rep_v)

        it16 = lax.iota(jnp.int32, 16)

        def wave(m, carry):
            pltpu.sync_copy(gh_s.at[m], ghb_v)
            mb = jnp.full((16,), m, jnp.int32)
            kvec = nm_v[m, :]
            av = av_v[m, :]
            avm = jnp.where(kvec < mb, av, 0.0)
            uvals = plsc.load_gather(rep_v, [kvec, mb])
            um = jnp.where(avm > 0.0, uvals, 0.0)

            racc = [jnp.zeros((16,), jnp.float32) for _ in range(8)]
            cacc = [jnp.zeros((16,), jnp.float32) for _ in range(8)]
            for s16 in range(_K):
                sb = jnp.full((16,), s16, jnp.int32)
                kb = kvec.at[sb].get(mode="promise_in_bounds")
                ab = avm.at[sb].get(mode="promise_in_bounds")
                ub = um.at[sb].get(mode="promise_in_bounds")
                for c8 in range(8):
                    ci = it16 + (16 * c8)
                    rv = plsc.load_gather(rep_v, [kb, ci])
                    mv = plsc.load_gather(mt_v, [kb, ci])
                    racc[c8] = racc[c8] + ab * (rv * mv)
                    avx = plsc.load_gather(at_v, [kb, ci])
                    cacc[c8] = cacc[c8] + ub * avx

            for c8 in range(8):
                off = 16 * c8
                ci = it16 + off
                vrow = ghb_v[0, pl.ds(off, 16)] + ghb_v[1, pl.ds(off, 16)] * racc[c8]
                vrow = jnp.minimum(jnp.maximum(vrow, 0.0), 1.0)
                plsc.store_scatter(rep_v, [mb, ci], vrow, mask=ci >= mb)
                vcol = ghb_v[2, pl.ds(off, 16)] + ghb_v[3, pl.ds(off, 16)] * cacc[c8]
                vcol = jnp.minimum(jnp.maximum(vcol, 0.0), 1.0)
                plsc.store_scatter(rep_v, [ci, mb], vcol, mask=ci > mb)
            return carry

        lax.fori_loop(0, _N, wave, 0)
        pltpu.sync_copy(rep_v, out_h)


def _sc_recurrence(GH, AT, MT, nm, AV, Z):
    f32 = jnp.float32
    mesh = plsc.VectorSubcoreMesh(core_axis_name="c", subcore_axis_name="s")
    fn = functools.partial(
        pl.kernel,
        mesh=mesh,
        compiler_params=pltpu.CompilerParams(needs_layout_passes=False),
        out_type=jax.ShapeDtypeStruct((_N, _N), f32),
        scratch_types=[
            pltpu.VMEM((_N, _N), f32),
            pltpu.VMEM((_N, _N), f32),
            pltpu.VMEM((_N, _K), jnp.int32),
            pltpu.VMEM((_N, _K), f32),
            pltpu.VMEM((_N, _N), f32),
            pltpu.VMEM((4, _N), f32),
            pltpu.VMEM_SHARED((_N, 4, _N), f32),
        ],
    )(_sc_body)
    return fn(GH, AT, MT, nm, AV, Z)


def kernel(dynamic_re, static_re, neighbor_matrix):
    G, H, GT, HT, AT, MT, AV = _precompute(dynamic_re, static_re, neighbor_matrix)
    GH = jnp.stack([G, H, GT, HT], axis=1)
    Z = jnp.zeros((_N, _N), jnp.float32)
    return _sc_recurrence(GH, AT, MT, neighbor_matrix, AV, Z)


# speedup vs baseline: 1.3589x; 1.3589x over previous
import functools

import jax
import jax.numpy as jnp
from jax import lax
from jax.experimental import pallas as pl
from jax.experimental.pallas import tpu as pltpu
from jax.experimental.pallas import tpu_sc as plsc

_N = 128
_K = 16


def _pre_body(d_ref, dt_ref, sr_ref, sc_ref, nm_ref, nmt_ref,
              g_ref, h_ref, gt_ref, ht_ref, at_ref, mt_ref, av_ref):
    n = _N
    D = d_ref[...]
    DT = dt_ref[...]
    s_row = sr_ref[...]
    s_col = sc_ref[...]
    ii = lax.broadcasted_iota(jnp.int32, (n, n), 0)
    jj = lax.broadcasted_iota(jnp.int32, (n, n), 1)
    diag = ii == jj

    M = jnp.zeros((n, n), jnp.bool_)
    MT = jnp.zeros((n, n), jnp.bool_)
    for k in range(_K):
        col_k = nm_ref[:, k][:, None]
        row_k = nmt_ref[k, :][None, :]
        M = M | (col_k == jj)
        MT = MT | (row_k == ii)
    Mf = M.astype(jnp.float32)
    MTf = MT.astype(jnp.float32)

    NB = jnp.tanh((0.7 * D + 0.3 * s_row + 0.5) * 0.5)
    NBT = jnp.tanh((0.7 * DT + 0.3 * s_col + 0.5) * 0.5)
    CNT = jnp.dot(Mf, MTf, precision=lax.Precision.HIGHEST)
    cnt_pos = CNT > 0.0
    hval = jnp.where(cnt_pos, 0.8 / jnp.maximum(CNT, 1.0), 0.0)

    A = Mf * jnp.where(diag, 1.0, NB)
    at_ref[...] = MTf * jnp.where(diag, 1.0, NBT)
    mt_ref[...] = MTf

    F = jnp.broadcast_to(0.5 * s_row, (n, n))
    FT = jnp.broadcast_to(0.5 * s_col, (n, n))
    g_ref[...] = jnp.where(diag, 1.0, jnp.where(M, NB, jnp.where(cnt_pos, 0.0, F)))
    gt_ref[...] = jnp.where(diag, 1.0, jnp.where(MT, NBT, jnp.where(cnt_pos, 0.0, FT)))
    h_ref[...] = jnp.where(diag | M, 0.0, hval)
    ht_ref[...] = jnp.where(diag | MT, 0.0, hval)

    av_cols = []
    for k in range(_K):
        col_k = nm_ref[:, k][:, None]
        dup = jnp.zeros((n, 1), jnp.bool_)
        for t in range(k):
            dup = dup | (nm_ref[:, t][:, None] == col_k)
        hit = (col_k == jj).astype(jnp.float32)
        val = jnp.sum(A * hit, axis=1, keepdims=True)
        av_cols.append(jnp.where(dup, 0.0, val))
    av_ref[...] = jnp.concatenate(av_cols, axis=1)


def _precompute(dynamic_re, static_re, neighbor_matrix):
    f32 = jnp.float32
    shp = jax.ShapeDtypeStruct
    return pl.pallas_call(
        _pre_body,
        out_shape=(
            shp((_N, _N), f32),
            shp((_N, _N), f32),
            shp((_N, _N), f32),
            shp((_N, _N), f32),
            shp((_N, _N), f32),
            shp((_N, _N), f32),
            shp((_N, _K), f32),
        ),
    )(dynamic_re, dynamic_re.T, static_re.reshape(1, _N),
      static_re.reshape(_N, 1), neighbor_matrix, neighbor_matrix.T)


def _sc_body(ghm_h, at_h, nm_h, av_h, z_h, out_h,
             at_v, b_v, nm_v, av_v, rep_v, ghb_v, ghm_s):
    cid = lax.axis_index("c")
    sid = lax.axis_index("s")

    @pl.when(jnp.logical_and(cid == 0, sid == 0))
    def _work():
        pltpu.sync_copy(ghm_h, ghm_s)
        pltpu.sync_copy(at_h, at_v)
        pltpu.sync_copy(nm_h, nm_v)
        pltpu.sync_copy(av_h, av_v)
        pltpu.sync_copy(z_h, rep_v)
        pltpu.sync_copy(z_h, b_v)

        it16 = lax.iota(jnp.int32, 16)

        def wave(m, carry):
            pltpu.sync_copy(ghm_s.at[m], ghb_v)
            mb = jnp.full((16,), m, jnp.int32)
            kvec = nm_v[m, :]
            av = av_v[m, :]
            avm = jnp.where(kvec < mb, av, 0.0)
            uvals = plsc.load_gather(rep_v, [kvec, mb])
            um = jnp.where(avm > 0.0, uvals, 0.0)

            racc = [jnp.zeros((16,), jnp.float32) for _ in range(8)]
            cacc = [jnp.zeros((16,), jnp.float32) for _ in range(8)]
            for s16 in range(_K):
                sb = jnp.full((16,), s16, jnp.int32)
                kb = kvec.at[sb].get(mode="promise_in_bounds")
                ab = avm.at[sb].get(mode="promise_in_bounds")
                ub = um.at[sb].get(mode="promise_in_bounds")
                for c8 in range(8):
                    ci = it16 + (16 * c8)
                    racc[c8] = racc[c8] + ab * plsc.load_gather(b_v, [kb, ci])
                    cacc[c8] = cacc[c8] + ub * plsc.load_gather(at_v, [kb, ci])

            for c8 in range(8):
                off = 16 * c8
                ci = it16 + off
                vrow = ghb_v[0, pl.ds(off, 16)] + ghb_v[1, pl.ds(off, 16)] * racc[c8]
                vrow = jnp.minimum(jnp.maximum(vrow, 0.0), 1.0)
                plsc.store_scatter(rep_v, [mb, ci], vrow, mask=ci >= mb)
                vcol = ghb_v[2, pl.ds(off, 16)] + ghb_v[3, pl.ds(off, 16)] * cacc[c8]
                vcol = jnp.minimum(jnp.maximum(vcol, 0.0), 1.0)
                plsc.store_scatter(rep_v, [ci, mb], vcol, mask=ci > mb)
            for c8 in range(8):
                off = 16 * c8
                b_v[m, pl.ds(off, 16)] = rep_v[m, pl.ds(off, 16)] * ghb_v[4, pl.ds(off, 16)]
            return carry

        lax.fori_loop(0, _N, wave, 0)
        pltpu.sync_copy(rep_v, out_h)


def _sc_recurrence(GHM, AT, nm, AV, Z):
    f32 = jnp.float32
    mesh = plsc.VectorSubcoreMesh(core_axis_name="c", subcore_axis_name="s")
    fn = functools.partial(
        pl.kernel,
        mesh=mesh,
        compiler_params=pltpu.CompilerParams(needs_layout_passes=False),
        out_type=jax.ShapeDtypeStruct((_N, _N), f32),
        scratch_types=[
            pltpu.VMEM((_N, _N), f32),
            pltpu.VMEM((_N, _N), f32),
            pltpu.VMEM((_N, _K), jnp.int32),
            pltpu.VMEM((_N, _K), f32),
            pltpu.VMEM((_N, _N), f32),
            pltpu.VMEM((5, _N), f32),
            pltpu.VMEM_SHARED((_N, 5, _N), f32),
        ],
    )(_sc_body)
    return fn(GHM, AT, nm, AV, Z)


def kernel(dynamic_re, static_re, neighbor_matrix):
    G, H, GT, HT, AT, MT, AV = _precompute(dynamic_re, static_re, neighbor_matrix)
    GHM = jnp.stack([G, H, GT, HT, MT], axis=1)
    Z = jnp.zeros((_N, _N), jnp.float32)
    return _sc_recurrence(GHM, AT, neighbor_matrix, AV, Z)


# speedup vs baseline: 2.3033x; 1.6949x over previous
import functools

import jax
import jax.numpy as jnp
from jax import lax
from jax.experimental import pallas as pl
from jax.experimental.pallas import tpu as pltpu
from jax.experimental.pallas import tpu_sc as plsc

_N = 128
_K = 16


def _pre_body(d_ref, dt_ref, sr_ref, sc_ref, nm_ref, nmt_ref,
              g_ref, h_ref, gt_ref, ht_ref, at_ref, mt_ref,
              kc_ref, avc_ref, cnt_ref):
    n = _N
    D = d_ref[...]
    DT = dt_ref[...]
    s_row = sr_ref[...]
    s_col = sc_ref[...]
    ii = lax.broadcasted_iota(jnp.int32, (n, n), 0)
    jj = lax.broadcasted_iota(jnp.int32, (n, n), 1)
    diag = ii == jj

    M = jnp.zeros((n, n), jnp.bool_)
    MT = jnp.zeros((n, n), jnp.bool_)
    for k in range(_K):
        col_k = nm_ref[:, k][:, None]
        row_k = nmt_ref[k, :][None, :]
        M = M | (col_k == jj)
        MT = MT | (row_k == ii)
    Mf = M.astype(jnp.float32)
    MTf = MT.astype(jnp.float32)

    NB = jnp.tanh((0.7 * D + 0.3 * s_row + 0.5) * 0.5)
    NBT = jnp.tanh((0.7 * DT + 0.3 * s_col + 0.5) * 0.5)
    CNT = jnp.dot(Mf, MTf, precision=lax.Precision.HIGHEST)
    cnt_pos = CNT > 0.0
    hval = jnp.where(cnt_pos, 0.8 / jnp.maximum(CNT, 1.0), 0.0)

    A = Mf * jnp.where(diag, 1.0, NB)
    at_ref[...] = MTf * jnp.where(diag, 1.0, NBT)
    mt_ref[...] = MTf

    F = jnp.broadcast_to(0.5 * s_row, (n, n))
    FT = jnp.broadcast_to(0.5 * s_col, (n, n))
    g_ref[...] = jnp.where(diag, 1.0, jnp.where(M, NB, jnp.where(cnt_pos, 0.0, F)))
    gt_ref[...] = jnp.where(diag, 1.0, jnp.where(MT, NBT, jnp.where(cnt_pos, 0.0, FT)))
    h_ref[...] = jnp.where(diag | M, 0.0, hval)
    ht_ref[...] = jnp.where(diag | MT, 0.0, hval)

    row_iota = lax.broadcasted_iota(jnp.int32, (n, 1), 0)
    acts, vals, ids = [], [], []
    for k in range(_K):
        col_k = nm_ref[:, k][:, None]
        dup = jnp.zeros((n, 1), jnp.bool_)
        for t in range(k):
            dup = dup | (nm_ref[:, t][:, None] == col_k)
        hit = (col_k == jj).astype(jnp.float32)
        vals.append(jnp.sum(A * hit, axis=1, keepdims=True))
        acts.append((~dup) & (col_k < row_iota))
        ids.append(col_k)
    run = jnp.zeros((n, 1), jnp.int32)
    ranks = []
    for k in range(_K):
        ranks.append(run)
        run = run + acts[k].astype(jnp.int32)
    kc_cols, avc_cols = [], []
    for p in range(_K):
        kcp = jnp.zeros((n, 1), jnp.int32)
        avp = jnp.zeros((n, 1), jnp.float32)
        for k in range(_K):
            sel = acts[k] & (ranks[k] == p)
            kcp = jnp.where(sel, ids[k], kcp)
            avp = jnp.where(sel, vals[k], avp)
        kc_cols.append(kcp)
        avc_cols.append(avp)
    kc_ref[...] = jnp.concatenate(kc_cols, axis=1)
    avc_ref[...] = jnp.concatenate(avc_cols, axis=1)
    cnt_ref[...] = jnp.broadcast_to(run, (n, _K))


def _precompute(dynamic_re, static_re, neighbor_matrix):
    f32 = jnp.float32
    shp = jax.ShapeDtypeStruct
    return pl.pallas_call(
        _pre_body,
        out_shape=(
            shp((_N, _N), f32),
            shp((_N, _N), f32),
            shp((_N, _N), f32),
            shp((_N, _N), f32),
            shp((_N, _N), f32),
            shp((_N, _N), f32),
            shp((_N, _K), jnp.int32),
            shp((_N, _K), f32),
            shp((_N, _K), jnp.int32),
        ),
    )(dynamic_re, dynamic_re.T, static_re.reshape(1, _N),
      static_re.reshape(_N, 1), neighbor_matrix, neighbor_matrix.T)


def _sc_body(ghm_h, at_h, kc_h, avc_h, cnt_h, z_h, out_h,
             at_v, b_v, kc_v, avc_v, cnt_v, rep_v, ghb_v, ghm_s):
    cid = lax.axis_index("c")
    sid = lax.axis_index("s")

    @pl.when(jnp.logical_and(cid == 0, sid == 0))
    def _work():
        pltpu.sync_copy(ghm_h, ghm_s)
        pltpu.sync_copy(at_h, at_v)
        pltpu.sync_copy(kc_h, kc_v)
        pltpu.sync_copy(avc_h, avc_v)
        pltpu.sync_copy(cnt_h, cnt_v)
        pltpu.sync_copy(z_h, rep_v)
        pltpu.sync_copy(z_h, b_v)

        it16 = lax.iota(jnp.int32, 16)
        zed = jnp.zeros((16,), jnp.float32)

        def wave(m, carry):
            pltpu.sync_copy(ghm_s.at[m], ghb_v)
            mb = jnp.full((16,), m, jnp.int32)
            kvec = kc_v[m, :]
            avv = avc_v[m, :]
            ns = jnp.max(cnt_v[m, :])

            def s_step(s, accs):
                sb = jnp.full((16,), s, jnp.int32)
                kb = kvec.at[sb].get(mode="promise_in_bounds")
                ab = avv.at[sb].get(mode="promise_in_bounds")
                ub = plsc.load_gather(rep_v, [kb, mb])
                out = []
                for c8 in range(8):
                    ci = it16 + (16 * c8)
                    out.append(accs[c8] + ab * plsc.load_gather(b_v, [kb, ci]))
                for c8 in range(8):
                    ci = it16 + (16 * c8)
                    out.append(accs[8 + c8] + ub * plsc.load_gather(at_v, [kb, ci]))
                return tuple(out)

            accs = lax.fori_loop(0, ns, s_step, tuple([zed] * 16))
            racc = accs[:8]
            cacc = accs[8:]

            for c8 in range(8):
                off = 16 * c8
                ci = it16 + off
                vrow = ghb_v[0, pl.ds(off, 16)] + ghb_v[1, pl.ds(off, 16)] * racc[c8]
                vrow = jnp.minimum(jnp.maximum(vrow, 0.0), 1.0)
                plsc.store_scatter(rep_v, [mb, ci], vrow, mask=ci >= mb)
                vcol = ghb_v[2, pl.ds(off, 16)] + ghb_v[3, pl.ds(off, 16)] * cacc[c8]
                vcol = jnp.minimum(jnp.maximum(vcol, 0.0), 1.0)
                plsc.store_scatter(rep_v, [ci, mb], vcol, mask=ci > mb)
            for c8 in range(8):
                off = 16 * c8
                b_v[m, pl.ds(off, 16)] = rep_v[m, pl.ds(off, 16)] * ghb_v[4, pl.ds(off, 16)]
            return carry

        lax.fori_loop(0, _N, wave, 0)
        pltpu.sync_copy(rep_v, out_h)


def _sc_recurrence(GHM, AT, KC, AVC, CNT, Z):
    f32 = jnp.float32
    mesh = plsc.VectorSubcoreMesh(core_axis_name="c", subcore_axis_name="s")
    fn = functools.partial(
        pl.kernel,
        mesh=mesh,
        compiler_params=pltpu.CompilerParams(needs_layout_passes=False),
        out_type=jax.ShapeDtypeStruct((_N, _N), f32),
        scratch_types=[
            pltpu.VMEM((_N, _N), f32),
            pltpu.VMEM((_N, _N), f32),
            pltpu.VMEM((_N, _K), jnp.int32),
            pltpu.VMEM((_N, _K), f32),
            pltpu.VMEM((_N, _K), jnp.int32),
            pltpu.VMEM((_N, _N), f32),
            pltpu.VMEM((5, _N), f32),
            pltpu.VMEM_SHARED((_N, 5, _N), f32),
        ],
    )(_sc_body)
    return fn(GHM, AT, KC, AVC, CNT, Z)


def kernel(dynamic_re, static_re, neighbor_matrix):
    G, H, GT, HT, AT, MT, KC, AVC, CNT = _precompute(
        dynamic_re, static_re, neighbor_matrix)
    GHM = jnp.stack([G, H, GT, HT, MT], axis=1)
    Z = jnp.zeros((_N, _N), jnp.float32)
    return _sc_recurrence(GHM, AT, KC, AVC, CNT, Z)


# speedup vs baseline: 2.6974x; 1.1711x over previous
import functools

import jax
import jax.numpy as jnp
from jax import lax
from jax.experimental import pallas as pl
from jax.experimental.pallas import tpu as pltpu
from jax.experimental.pallas import tpu_sc as plsc

_N = 128
_K = 16


def _pre_body(d_ref, dt_ref, sr_ref, sc_ref, nm_ref, nmt_ref,
              g_ref, h_ref, gt_ref, ht_ref, at_ref, mt_ref,
              kc_ref, avc_ref, cnt_ref):
    n = _N
    D = d_ref[...]
    DT = dt_ref[...]
    s_row = sr_ref[...]
    s_col = sc_ref[...]
    ii = lax.broadcasted_iota(jnp.int32, (n, n), 0)
    jj = lax.broadcasted_iota(jnp.int32, (n, n), 1)
    diag = ii == jj

    M = jnp.zeros((n, n), jnp.bool_)
    MT = jnp.zeros((n, n), jnp.bool_)
    for k in range(_K):
        col_k = nm_ref[:, k][:, None]
        row_k = nmt_ref[k, :][None, :]
        M = M | (col_k == jj)
        MT = MT | (row_k == ii)
    Mf = M.astype(jnp.float32)
    MTf = MT.astype(jnp.float32)

    NB = jnp.tanh((0.7 * D + 0.3 * s_row + 0.5) * 0.5)
    NBT = jnp.tanh((0.7 * DT + 0.3 * s_col + 0.5) * 0.5)
    CNT = jnp.dot(Mf, MTf, precision=lax.Precision.HIGHEST)
    cnt_pos = CNT > 0.0
    hval = jnp.where(cnt_pos, 0.8 / jnp.maximum(CNT, 1.0), 0.0)

    A = Mf * jnp.where(diag, 1.0, NB)
    at_ref[...] = MTf * jnp.where(diag, 1.0, NBT)
    mt_ref[...] = MTf

    F = jnp.broadcast_to(0.5 * s_row, (n, n))
    FT = jnp.broadcast_to(0.5 * s_col, (n, n))
    g_ref[...] = jnp.where(diag, 1.0, jnp.where(M, NB, jnp.where(cnt_pos, 0.0, F)))
    gt_ref[...] = jnp.where(diag, 1.0, jnp.where(MT, NBT, jnp.where(cnt_pos, 0.0, FT)))
    h_ref[...] = jnp.where(diag | M, 0.0, hval)
    ht_ref[...] = jnp.where(diag | MT, 0.0, hval)

    row_iota = lax.broadcasted_iota(jnp.int32, (n, 1), 0)
    acts, vals, ids = [], [], []
    for k in range(_K):
        col_k = nm_ref[:, k][:, None]
        dup = jnp.zeros((n, 1), jnp.bool_)
        for t in range(k):
            dup = dup | (nm_ref[:, t][:, None] == col_k)
        hit = (col_k == jj).astype(jnp.float32)
        vals.append(jnp.sum(A * hit, axis=1, keepdims=True))
        acts.append((~dup) & (col_k < row_iota))
        ids.append(col_k)
    run = jnp.zeros((n, 1), jnp.int32)
    ranks = []
    for k in range(_K):
        ranks.append(run)
        run = run + acts[k].astype(jnp.int32)
    kc_cols, avc_cols = [], []
    for p in range(_K):
        kcp = jnp.zeros((n, 1), jnp.int32)
        avp = jnp.zeros((n, 1), jnp.float32)
        for k in range(_K):
            sel = acts[k] & (ranks[k] == p)
            kcp = jnp.where(sel, ids[k], kcp)
            avp = jnp.where(sel, vals[k], avp)
        kc_cols.append(kcp)
        avc_cols.append(avp)
    kc_ref[...] = jnp.concatenate(kc_cols, axis=1)
    avc_ref[...] = jnp.concatenate(avc_cols, axis=1)
    cnt_ref[...] = jnp.broadcast_to(run, (n, _K))


def _precompute(dynamic_re, static_re, neighbor_matrix):
    f32 = jnp.float32
    shp = jax.ShapeDtypeStruct
    return pl.pallas_call(
        _pre_body,
        out_shape=(
            shp((_N, _N), f32),
            shp((_N, _N), f32),
            shp((_N, _N), f32),
            shp((_N, _N), f32),
            shp((_N, _N), f32),
            shp((_N, _N), f32),
            shp((_N, _K), jnp.int32),
            shp((_N, _K), f32),
            shp((_N, _K), jnp.int32),
        ),
    )(dynamic_re, dynamic_re.T, static_re.reshape(1, _N),
      static_re.reshape(_N, 1), neighbor_matrix, neighbor_matrix.T)


def _sc_body(ghm_h, at_h, kc_h, avc_h, cnt_h, z_h, out_h,
             at_v, b_v, kc_v, avc_v, cnt_v, rep_v, gh0_v, gh1_v, ghm_s, sem):
    cid = lax.axis_index("c")
    sid = lax.axis_index("s")

    @pl.when(jnp.logical_and(cid == 0, sid == 0))
    def _work():
        pltpu.sync_copy(ghm_h, ghm_s)
        pltpu.sync_copy(at_h, at_v)
        pltpu.sync_copy(kc_h, kc_v)
        pltpu.sync_copy(avc_h, avc_v)
        pltpu.sync_copy(cnt_h, cnt_v)
        pltpu.sync_copy(z_h, rep_v)
        pltpu.sync_copy(z_h, b_v)

        it16 = lax.iota(jnp.int32, 16)
        zed = jnp.zeros((16,), jnp.float32)

        pltpu.make_async_copy(ghm_s.at[0], gh0_v, sem).start()

        def wave(m, ghb_v, nxt, ghn_v):
            pltpu.make_async_copy(ghm_s.at[m], ghb_v, sem).wait()
            pltpu.make_async_copy(ghm_s.at[nxt], ghn_v, sem).start()
            mb = jnp.full((16,), m, jnp.int32)
            kvec = kc_v[m, :]
            avv = avc_v[m, :]
            ns = jnp.max(cnt_v[m, :])

            def s_step(s, accs):
                sb = jnp.full((16,), s, jnp.int32)
                kb = kvec.at[sb].get(mode="promise_in_bounds")
                ab = avv.at[sb].get(mode="promise_in_bounds")
                ub = plsc.load_gather(rep_v, [kb, mb])
                out = []
                for c8 in range(8):
                    ci = it16 + (16 * c8)
                    out.append(accs[c8] + ab * plsc.load_gather(b_v, [kb, ci]))
                for c8 in range(8):
                    ci = it16 + (16 * c8)
                    out.append(accs[8 + c8] + ub * plsc.load_gather(at_v, [kb, ci]))
                return tuple(out)

            accs = lax.fori_loop(0, ns, s_step, tuple([zed] * 16))
            racc = accs[:8]
            cacc = accs[8:]

            for c8 in range(8):
                off = 16 * c8
                ci = it16 + off
                vrow = ghb_v[0, pl.ds(off, 16)] + ghb_v[1, pl.ds(off, 16)] * racc[c8]
                vrow = jnp.minimum(jnp.maximum(vrow, 0.0), 1.0)
                old = rep_v[m, pl.ds(off, 16)]
                merged = jnp.where(ci >= mb, vrow, old)
                rep_v[m, pl.ds(off, 16)] = merged
                b_v[m, pl.ds(off, 16)] = merged * ghb_v[4, pl.ds(off, 16)]
                vcol = ghb_v[2, pl.ds(off, 16)] + ghb_v[3, pl.ds(off, 16)] * cacc[c8]
                vcol = jnp.minimum(jnp.maximum(vcol, 0.0), 1.0)
                plsc.store_scatter(rep_v, [ci, mb], vcol, mask=ci > mb)

        def wave_pair(t, carry):
            m0 = 2 * t
            wave(m0, gh0_v, m0 + 1, gh1_v)
            wave(m0 + 1, gh1_v, jnp.minimum(m0 + 2, _N - 1), gh0_v)
            return carry

        lax.fori_loop(0, _N // 2, wave_pair, 0)
        pltpu.make_async_copy(ghm_s.at[0], gh0_v, sem).wait()
        pltpu.sync_copy(rep_v, out_h)


def _sc_recurrence(GHM, AT, KC, AVC, CNT, Z):
    f32 = jnp.float32
    mesh = plsc.VectorSubcoreMesh(core_axis_name="c", subcore_axis_name="s")
    fn = functools.partial(
        pl.kernel,
        mesh=mesh,
        compiler_params=pltpu.CompilerParams(needs_layout_passes=False),
        out_type=jax.ShapeDtypeStruct((_N, _N), f32),
        scratch_types=[
            pltpu.VMEM((_N, _N), f32),
            pltpu.VMEM((_N, _N), f32),
            pltpu.VMEM((_N, _K), jnp.int32),
            pltpu.VMEM((_N, _K), f32),
            pltpu.VMEM((_N, _K), jnp.int32),
            pltpu.VMEM((_N, _N), f32),
            pltpu.VMEM((5, _N), f32),
            pltpu.VMEM((5, _N), f32),
            pltpu.VMEM_SHARED((_N, 5, _N), f32),
            pltpu.SemaphoreType.DMA,
        ],
    )(_sc_body)
    return fn(GHM, AT, KC, AVC, CNT, Z)


def kernel(dynamic_re, static_re, neighbor_matrix):
    G, H, GT, HT, AT, MT, KC, AVC, CNT = _precompute(
        dynamic_re, static_re, neighbor_matrix)
    GHM = jnp.stack([G, H, GT, HT, MT], axis=1)
    Z = jnp.zeros((_N, _N), jnp.float32)
    return _sc_recurrence(GHM, AT, KC, AVC, CNT, Z)


# speedup vs baseline: 3.0891x; 1.1452x over previous
import functools

import jax
import jax.numpy as jnp
from jax import lax
from jax.experimental import pallas as pl
from jax.experimental.pallas import tpu as pltpu
from jax.experimental.pallas import tpu_sc as plsc

_N = 128
_K = 16


def _pre_body(d_ref, dt_ref, sr_ref, sc_ref, nm_ref, nmt_ref,
              g_ref, h_ref, gt_ref, ht_ref, at_ref, mt_ref,
              kc_ref, avc_ref, cnt_ref):
    n = _N
    D = d_ref[...]
    DT = dt_ref[...]
    s_row = sr_ref[...]
    s_col = sc_ref[...]
    ii = lax.broadcasted_iota(jnp.int32, (n, n), 0)
    jj = lax.broadcasted_iota(jnp.int32, (n, n), 1)
    diag = ii == jj

    M = jnp.zeros((n, n), jnp.bool_)
    MT = jnp.zeros((n, n), jnp.bool_)
    for k in range(_K):
        col_k = nm_ref[:, k][:, None]
        row_k = nmt_ref[k, :][None, :]
        M = M | (col_k == jj)
        MT = MT | (row_k == ii)
    Mf = M.astype(jnp.float32)
    MTf = MT.astype(jnp.float32)

    NB = jnp.tanh((0.7 * D + 0.3 * s_row + 0.5) * 0.5)
    NBT = jnp.tanh((0.7 * DT + 0.3 * s_col + 0.5) * 0.5)
    CNT = jnp.dot(Mf, MTf, precision=lax.Precision.HIGHEST)
    cnt_pos = CNT > 0.0
    hval = jnp.where(cnt_pos, 0.8 / jnp.maximum(CNT, 1.0), 0.0)

    A = Mf * jnp.where(diag, 1.0, NB)
    at_ref[...] = MTf * jnp.where(diag, 1.0, NBT)
    mt_ref[...] = MTf

    F = jnp.broadcast_to(0.5 * s_row, (n, n))
    FT = jnp.broadcast_to(0.5 * s_col, (n, n))
    g_ref[...] = jnp.where(diag, 1.0, jnp.where(M, NB, jnp.where(cnt_pos, 0.0, F)))
    gt_ref[...] = jnp.where(diag, 1.0, jnp.where(MT, NBT, jnp.where(cnt_pos, 0.0, FT)))
    h_ref[...] = jnp.where(diag | M, 0.0, hval)
    ht_ref[...] = jnp.where(diag | MT, 0.0, hval)

    row_iota = lax.broadcasted_iota(jnp.int32, (n, 1), 0)
    acts, vals, ids = [], [], []
    for k in range(_K):
        col_k = nm_ref[:, k][:, None]
        dup = jnp.zeros((n, 1), jnp.bool_)
        for t in range(k):
            dup = dup | (nm_ref[:, t][:, None] == col_k)
        hit = (col_k == jj).astype(jnp.float32)
        vals.append(jnp.sum(A * hit, axis=1, keepdims=True))
        acts.append((~dup) & (col_k < row_iota))
        ids.append(col_k)
    run = jnp.zeros((n, 1), jnp.int32)
    ranks = []
    for k in range(_K):
        ranks.append(run)
        run = run + acts[k].astype(jnp.int32)
    kc_cols, avc_cols = [], []
    for p in range(_K):
        kcp = jnp.zeros((n, 1), jnp.int32)
        avp = jnp.zeros((n, 1), jnp.float32)
        for k in range(_K):
            sel = acts[k] & (ranks[k] == p)
            kcp = jnp.where(sel, ids[k], kcp)
            avp = jnp.where(sel, vals[k], avp)
        kc_cols.append(kcp)
        avc_cols.append(avp)
    kc_ref[...] = jnp.concatenate(kc_cols, axis=1)
    avc_ref[...] = jnp.concatenate(avc_cols, axis=1)
    cnt_ref[...] = jnp.broadcast_to(run, (n, _K))


def _precompute(dynamic_re, static_re, neighbor_matrix):
    f32 = jnp.float32
    shp = jax.ShapeDtypeStruct
    return pl.pallas_call(
        _pre_body,
        out_shape=(
            shp((_N, _N), f32),
            shp((_N, _N), f32),
            shp((_N, _N), f32),
            shp((_N, _N), f32),
            shp((_N, _N), f32),
            shp((_N, _N), f32),
            shp((_N, _K), jnp.int32),
            shp((_N, _K), f32),
            shp((_N, _K), jnp.int32),
        ),
    )(dynamic_re, dynamic_re.T, static_re.reshape(1, _N),
      static_re.reshape(_N, 1), neighbor_matrix, neighbor_matrix.T)


def _sc_body(ghm_h, at_h, kc_h, avc_h, cnt_h, z_h, out_h,
             at_v, b_v, kc_v, avc_v, cnt_v, rep_v, gh0_v, gh1_v, ghm_s, sem):
    cid = lax.axis_index("c")
    sid = lax.axis_index("s")

    @pl.when(jnp.logical_and(cid == 0, sid == 0))
    def _work():
        pltpu.sync_copy(ghm_h, ghm_s)
        pltpu.sync_copy(at_h, at_v)
        pltpu.sync_copy(kc_h, kc_v)
        pltpu.sync_copy(avc_h, avc_v)
        pltpu.sync_copy(cnt_h, cnt_v)
        pltpu.sync_copy(z_h, rep_v)
        pltpu.sync_copy(z_h, b_v)

        it16 = lax.iota(jnp.int32, 16)
        zed = jnp.zeros((16,), jnp.float32)

        pltpu.make_async_copy(ghm_s.at[0], gh0_v, sem).start()

        def wave(m, ghb_v, nxt, ghn_v, clo):
            pltpu.make_async_copy(ghm_s.at[m], ghb_v, sem).wait()
            pltpu.make_async_copy(ghm_s.at[nxt], ghn_v, sem).start()
            mb = jnp.full((16,), m, jnp.int32)
            kvec = kc_v[m, :]
            avv = avc_v[m, :]
            ns = jnp.max(cnt_v[m, :])
            nch = 8 - clo

            def s_step(s, accs):
                sb = jnp.full((16,), s, jnp.int32)
                kb = kvec.at[sb].get(mode="promise_in_bounds")
                ab = avv.at[sb].get(mode="promise_in_bounds")
                ub = plsc.load_gather(rep_v, [kb, mb])
                out = []
                for c8 in range(clo, 8):
                    ci = it16 + (16 * c8)
                    out.append(accs[c8 - clo] + ab * plsc.load_gather(b_v, [kb, ci]))
                for c8 in range(clo, 8):
                    ci = it16 + (16 * c8)
                    out.append(accs[nch + c8 - clo] + ub * plsc.load_gather(at_v, [kb, ci]))
                return tuple(out)

            accs = lax.fori_loop(0, ns, s_step, tuple([zed] * (2 * nch)))
            racc = accs[:nch]
            cacc = accs[nch:]

            for c8 in range(clo, 8):
                off = 16 * c8
                ci = it16 + off
                vrow = ghb_v[0, pl.ds(off, 16)] + ghb_v[1, pl.ds(off, 16)] * racc[c8 - clo]
                vrow = jnp.minimum(jnp.maximum(vrow, 0.0), 1.0)
                old = rep_v[m, pl.ds(off, 16)]
                merged = jnp.where(ci >= mb, vrow, old)
                rep_v[m, pl.ds(off, 16)] = merged
                b_v[m, pl.ds(off, 16)] = merged * ghb_v[4, pl.ds(off, 16)]
                vcol = ghb_v[2, pl.ds(off, 16)] + ghb_v[3, pl.ds(off, 16)] * cacc[c8 - clo]
                vcol = jnp.minimum(jnp.maximum(vcol, 0.0), 1.0)
                plsc.store_scatter(rep_v, [ci, mb], vcol, mask=ci > mb)

        def make_pair(clo):
            def wave_pair(t, carry):
                m0 = 2 * t
                wave(m0, gh0_v, m0 + 1, gh1_v, clo)
                wave(m0 + 1, gh1_v, jnp.minimum(m0 + 2, _N - 1), gh0_v, clo)
                return carry
            return wave_pair

        for v in range(4):
            lax.fori_loop(16 * v, 16 * (v + 1), make_pair(2 * v), 0)
        pltpu.make_async_copy(ghm_s.at[0], gh0_v, sem).wait()
        pltpu.sync_copy(rep_v, out_h)


def _sc_recurrence(GHM, AT, KC, AVC, CNT, Z):
    f32 = jnp.float32
    mesh = plsc.VectorSubcoreMesh(core_axis_name="c", subcore_axis_name="s")
    fn = functools.partial(
        pl.kernel,
        mesh=mesh,
        compiler_params=pltpu.CompilerParams(needs_layout_passes=False),
        out_type=jax.ShapeDtypeStruct((_N, _N), f32),
        scratch_types=[
            pltpu.VMEM((_N, _N), f32),
            pltpu.VMEM((_N, _N), f32),
            pltpu.VMEM((_N, _K), jnp.int32),
            pltpu.VMEM((_N, _K), f32),
            pltpu.VMEM((_N, _K), jnp.int32),
            pltpu.VMEM((_N, _N), f32),
            pltpu.VMEM((5, _N), f32),
            pltpu.VMEM((5, _N), f32),
            pltpu.VMEM_SHARED((_N, 5, _N), f32),
            pltpu.SemaphoreType.DMA,
        ],
    )(_sc_body)
    return fn(GHM, AT, KC, AVC, CNT, Z)


def kernel(dynamic_re, static_re, neighbor_matrix):
    G, H, GT, HT, AT, MT, KC, AVC, CNT = _precompute(
        dynamic_re, static_re, neighbor_matrix)
    GHM = jnp.stack([G, H, GT, HT, MT], axis=1)
    Z = jnp.zeros((_N, _N), jnp.float32)
    return _sc_recurrence(GHM, AT, KC, AVC, CNT, Z)


# speedup vs baseline: 3.3740x; 1.0922x over previous
import functools

import jax
import jax.numpy as jnp
from jax import lax
from jax.experimental import pallas as pl
from jax.experimental.pallas import tpu as pltpu
from jax.experimental.pallas import tpu_sc as plsc

_N = 128
_K = 16


def _pre_body(d_ref, dt_ref, sr_ref, sc_ref, nm_ref, nmt_ref,
              g_ref, h_ref, gt_ref, ht_ref, at_ref, mt_ref,
              kc_ref, avc_ref, cnt_ref):
    n = _N
    D = d_ref[...]
    DT = dt_ref[...]
    s_row = sr_ref[...]
    s_col = sc_ref[...]
    ii = lax.broadcasted_iota(jnp.int32, (n, n), 0)
    jj = lax.broadcasted_iota(jnp.int32, (n, n), 1)
    diag = ii == jj

    M = jnp.zeros((n, n), jnp.bool_)
    MT = jnp.zeros((n, n), jnp.bool_)
    for k in range(_K):
        col_k = nm_ref[:, k][:, None]
        row_k = nmt_ref[k, :][None, :]
        M = M | (col_k == jj)
        MT = MT | (row_k == ii)
    Mf = M.astype(jnp.float32)
    MTf = MT.astype(jnp.float32)

    NB = jnp.tanh((0.7 * D + 0.3 * s_row + 0.5) * 0.5)
    NBT = jnp.tanh((0.7 * DT + 0.3 * s_col + 0.5) * 0.5)
    CNT = jnp.dot(Mf, MTf, precision=lax.Precision.HIGHEST)
    cnt_pos = CNT > 0.0
    hval = jnp.where(cnt_pos, 0.8 / jnp.maximum(CNT, 1.0), 0.0)

    A = Mf * jnp.where(diag, 1.0, NB)
    at_ref[...] = MTf * jnp.where(diag, 1.0, NBT)
    mt_ref[...] = MTf

    F = jnp.broadcast_to(0.5 * s_row, (n, n))
    FT = jnp.broadcast_to(0.5 * s_col, (n, n))
    g_ref[...] = jnp.where(diag, 1.0, jnp.where(M, NB, jnp.where(cnt_pos, 0.0, F)))
    gt_ref[...] = jnp.where(diag, 1.0, jnp.where(MT, NBT, jnp.where(cnt_pos, 0.0, FT)))
    h_ref[...] = jnp.where(diag | M, 0.0, hval)
    ht_ref[...] = jnp.where(diag | MT, 0.0, hval)

    row_iota = lax.broadcasted_iota(jnp.int32, (n, 1), 0)
    acts, vals, ids = [], [], []
    for k in range(_K):
        col_k = nm_ref[:, k][:, None]
        dup = jnp.zeros((n, 1), jnp.bool_)
        for t in range(k):
            dup = dup | (nm_ref[:, t][:, None] == col_k)
        hit = (col_k == jj).astype(jnp.float32)
        vals.append(jnp.sum(A * hit, axis=1, keepdims=True))
        acts.append((~dup) & (col_k < row_iota))
        ids.append(col_k)
    run = jnp.zeros((n, 1), jnp.int32)
    ranks = []
    for k in range(_K):
        ranks.append(run)
        run = run + acts[k].astype(jnp.int32)
    kc_cols, avc_cols = [], []
    for p in range(_K):
        kcp = jnp.zeros((n, 1), jnp.int32)
        avp = jnp.zeros((n, 1), jnp.float32)
        for k in range(_K):
            sel = acts[k] & (ranks[k] == p)
            kcp = jnp.where(sel, ids[k], kcp)
            avp = jnp.where(sel, vals[k], avp)
        kc_cols.append(kcp)
        avc_cols.append(avp)
    kc_ref[...] = jnp.concatenate(kc_cols, axis=1)
    avc_ref[...] = jnp.concatenate(avc_cols, axis=1)
    cnt_ref[...] = jnp.broadcast_to(run, (n, _K))


def _precompute(dynamic_re, static_re, neighbor_matrix):
    f32 = jnp.float32
    shp = jax.ShapeDtypeStruct
    return pl.pallas_call(
        _pre_body,
        out_shape=(
            shp((_N, _N), f32),
            shp((_N, _N), f32),
            shp((_N, _N), f32),
            shp((_N, _N), f32),
            shp((_N, _N), f32),
            shp((_N, _N), f32),
            shp((_N, _K), jnp.int32),
            shp((_N, _K), f32),
            shp((_N, _K), jnp.int32),
        ),
    )(dynamic_re, dynamic_re.T, static_re.reshape(1, _N),
      static_re.reshape(_N, 1), neighbor_matrix, neighbor_matrix.T)


def _sc_body(ghm_h, at_h, kc_h, avc_h, cnt_h, out_h,
             at_v, b_v, kc_v, avc_v, cnt_v, rep_v, gh0_v, gh1_v, ghm_s, sem):
    cid = lax.axis_index("c")
    sid = lax.axis_index("s")

    @pl.when(jnp.logical_and(cid == 0, sid == 0))
    def _work():
        cps = [
            pltpu.make_async_copy(ghm_h, ghm_s, sem),
            pltpu.make_async_copy(at_h, at_v, sem),
            pltpu.make_async_copy(kc_h, kc_v, sem),
            pltpu.make_async_copy(avc_h, avc_v, sem),
            pltpu.make_async_copy(cnt_h, cnt_v, sem),
        ]
        for cp in cps:
            cp.start()
        for cp in cps:
            cp.wait()

        it16 = lax.iota(jnp.int32, 16)
        zed = jnp.zeros((16,), jnp.float32)

        pltpu.make_async_copy(ghm_s.at[0], gh0_v, sem).start()

        def wave(m, ghb_v, nxt, ghn_v, clo):
            pltpu.make_async_copy(ghm_s.at[m], ghb_v, sem).wait()
            pltpu.make_async_copy(ghm_s.at[nxt], ghn_v, sem).start()
            mb = jnp.full((16,), m, jnp.int32)
            kvec = kc_v[m, :]
            avv = avc_v[m, :]
            ns = jnp.max(cnt_v[m, :])
            nch = 8 - clo

            def slot(s, padded):
                sb = jnp.full((16,), s, jnp.int32)
                kb = kvec.at[sb].get(mode="promise_in_bounds")
                ab = avv.at[sb].get(mode="promise_in_bounds")
                ub = plsc.load_gather(rep_v, [kb, mb])
                if padded:
                    ub = jnp.where(ab > 0.0, ub, 0.0)
                return kb, ab, ub

            def s_step(s2, accs):
                ka, aa, ua = slot(2 * s2, False)
                kb_, ab_, ub_ = slot(2 * s2 + 1, True)
                out = []
                for c8 in range(clo, 8):
                    ci = it16 + (16 * c8)
                    out.append(accs[c8 - clo]
                               + aa * plsc.load_gather(b_v, [ka, ci])
                               + ab_ * plsc.load_gather(b_v, [kb_, ci]))
                for c8 in range(clo, 8):
                    ci = it16 + (16 * c8)
                    out.append(accs[nch + c8 - clo]
                               + ua * plsc.load_gather(at_v, [ka, ci])
                               + ub_ * plsc.load_gather(at_v, [kb_, ci]))
                return tuple(out)

            accs = lax.fori_loop(0, (ns + 1) >> 1, s_step,
                                 tuple([zed] * (2 * nch)))
            racc = accs[:nch]
            cacc = accs[nch:]

            for c8 in range(clo, 8):
                off = 16 * c8
                ci = it16 + off
                vrow = ghb_v[0, pl.ds(off, 16)] + ghb_v[1, pl.ds(off, 16)] * racc[c8 - clo]
                vrow = jnp.minimum(jnp.maximum(vrow, 0.0), 1.0)
                old = rep_v[m, pl.ds(off, 16)]
                merged = jnp.where(ci >= mb, vrow, old)
                rep_v[m, pl.ds(off, 16)] = merged
                b_v[m, pl.ds(off, 16)] = merged * ghb_v[4, pl.ds(off, 16)]
                vcol = ghb_v[2, pl.ds(off, 16)] + ghb_v[3, pl.ds(off, 16)] * cacc[c8 - clo]
                vcol = jnp.minimum(jnp.maximum(vcol, 0.0), 1.0)
                plsc.store_scatter(rep_v, [ci, mb], vcol, mask=ci > mb)

        def make_pair(clo):
            def wave_pair(t, carry):
                m0 = 2 * t
                wave(m0, gh0_v, m0 + 1, gh1_v, clo)
                wave(m0 + 1, gh1_v, jnp.minimum(m0 + 2, _N - 1), gh0_v, clo)
                return carry
            return wave_pair

        for v in range(4):
            lax.fori_loop(16 * v, 16 * (v + 1), make_pair(2 * v), 0)
        pltpu.make_async_copy(ghm_s.at[0], gh0_v, sem).wait()
        pltpu.sync_copy(rep_v, out_h)


def _sc_recurrence(GHM, AT, KC, AVC, CNT):
    f32 = jnp.float32
    mesh = plsc.VectorSubcoreMesh(core_axis_name="c", subcore_axis_name="s")
    fn = functools.partial(
        pl.kernel,
        mesh=mesh,
        compiler_params=pltpu.CompilerParams(needs_layout_passes=False),
        out_type=jax.ShapeDtypeStruct((_N, _N), f32),
        scratch_types=[
            pltpu.VMEM((_N, _N), f32),
            pltpu.VMEM((_N, _N), f32),
            pltpu.VMEM((_N, _K), jnp.int32),
            pltpu.VMEM((_N, _K), f32),
            pltpu.VMEM((_N, _K), jnp.int32),
            pltpu.VMEM((_N, _N), f32),
            pltpu.VMEM((5, _N), f32),
            pltpu.VMEM((5, _N), f32),
            pltpu.VMEM_SHARED((_N, 5, _N), f32),
            pltpu.SemaphoreType.DMA,
        ],
    )(_sc_body)
    return fn(GHM, AT, KC, AVC, CNT)


def kernel(dynamic_re, static_re, neighbor_matrix):
    G, H, GT, HT, AT, MT, KC, AVC, CNT = _precompute(
        dynamic_re, static_re, neighbor_matrix)
    GHM = jnp.stack([G, H, GT, HT, MT], axis=1)
    return _sc_recurrence(GHM, AT, KC, AVC, CNT)
